# baseline (speedup 1.0000x reference)
import jax
import jax.numpy as jnp
import numpy as np
from jax import lax
from jax.experimental import pallas as pl

_N = 4096


def _butterfly_positions(n):
    indices_in = [[0, 0, 1, 1]]
    indices_out = [[0, 1, 0, 1]]
    curr_n = 2
    while curr_n < n:
        for i in range(len(indices_in)):
            indices_in[i] = indices_in[i] + [p + curr_n for p in indices_in[i]]
            indices_out[i] = indices_out[i] + [p + curr_n for p in indices_out[i]]
        sublist_low = list(range(curr_n)) * 2
        sublist_high = list(range(curr_n, curr_n * 2)) * 2
        new_idx_in = sublist_low + sublist_high
        indices_in.append(new_idx_in)
        new_idx_out = list(range(curr_n * 2)) * 2
        indices_out.append(new_idx_out)
        indices_in.insert(0, list(new_idx_in))
        indices_out.insert(0, list(new_idx_out))
        curr_n *= 2
    ii = np.array(indices_in)
    io = np.array(indices_out)
    L = ii.shape[0]
    pos_a = np.zeros((L, n), dtype=np.int64)
    pos_b = np.zeros((L, n), dtype=np.int64)
    strides = []
    for l in range(L):
        diag = ii[l] == io[l]
        pos_a[l, io[l, diag]] = np.nonzero(diag)[0]
        pos_b[l, io[l, ~diag]] = np.nonzero(~diag)[0]
        strides.append(int(abs(ii[l, ~diag][0] - io[l, ~diag][0])))
    return strides, pos_a, pos_b


_STRIDES, _POS_A, _POS_B = _butterfly_positions(_N)
_L = len(_STRIDES)
_FRONT = [l for l in range(_L) if _STRIDES[l] >= 128 and l < _L // 2]
_BACK = [l for l in range(_L) if _STRIDES[l] >= 128 and l > _L // 2]
_MID = [l for l in range(_L) if _STRIDES[l] < 128]
_BIG = _FRONT + _BACK


def _selection_constants():
    sa = np.zeros((len(_MID), 256, 128), dtype=np.float32)
    sb = np.zeros((len(_MID), 256, 128), dtype=np.float32)
    for t, l in enumerate(_MID):
        for s_mat, pos in ((sa, _POS_A), (sb, _POS_B)):
            p = pos[l].reshape(32, 128)
            off = p - 256 * np.arange(32)[:, None]
            assert (off == off[0]).all()
            s_mat[t, off[0], np.arange(128)] = 1.0
    pa = np.zeros((len(_BIG), 32, 64), dtype=np.float32)
    pb = np.zeros((len(_BIG), 32, 64), dtype=np.float32)
    for t, l in enumerate(_BIG):
        for p_mat, pos in ((pa, _POS_A), (pb, _POS_B)):
            p = pos[l].reshape(32, 128)
            assert (p % 128 == np.arange(128)[None, :]).all()
            p_mat[t, np.arange(32), p[:, 0] // 128] = 1.0
    return sa, sb, pa, pb


_SA, _SB, _PA, _PB = _selection_constants()


def _extract_ab(values):
    vmid = values[_MID[0]:_MID[-1] + 1].reshape(len(_MID), 32, 256)
    dn = (((2,), (1,)), ((0,), (0,)))
    hp = lax.Precision.HIGHEST
    a_mid = lax.dot_general(vmid, jnp.asarray(_SA), dn, precision=hp,
                            preferred_element_type=jnp.float32)
    b_mid = lax.dot_general(vmid, jnp.asarray(_SB), dn, precision=hp,
                            preferred_element_type=jnp.float32)
    vbig = jnp.concatenate(
        [values[:_MID[0]], values[_MID[-1] + 1:]], axis=0).reshape(
            len(_BIG), 64, 128)
    dn2 = (((2,), (1,)), ((0,), (0,)))
    a_big = lax.dot_general(jnp.asarray(_PA), vbig, dn2, precision=hp,
                            preferred_element_type=jnp.float32)
    b_big = lax.dot_general(jnp.asarray(_PB), vbig, dn2, precision=hp,
                            preferred_element_type=jnp.float32)
    return (a_mid.reshape(len(_MID), _N), b_mid.reshape(len(_MID), _N),
            a_big.reshape(len(_BIG), _N), b_big.reshape(len(_BIG), _N))


def _prep_body(amt_ref, bmt_ref, m_ref):
    row = lax.broadcasted_iota(jnp.int32, (_N, 128), 0)
    col = lax.broadcasted_iota(jnp.int32, (_N, 128), 1)
    m = jnp.where((row % 128) == col, 1.0, 0.0).astype(jnp.float32)
    for t, l in enumerate(_MID):
        s = _STRIDES[l]
        a = amt_ref[:, t:t + 1]
        b = bmt_ref[:, t:t + 1]
        take_lo = (row & s) == 0
        swapped = jnp.where(take_lo, jnp.roll(m, -s, axis=0),
                            jnp.roll(m, s, axis=0))
        m = a * m + b * swapped
    m_ref[...] = m


def _swap(x, s):
    n = x.shape[-1]
    parts = []
    for g in range(n // (2 * s)):
        parts.append(x[:, g * 2 * s + s:(g + 1) * 2 * s])
        parts.append(x[:, g * 2 * s:g * 2 * s + s])
    return jnp.concatenate(parts, axis=-1)


def _tile_body(x_ref, af_ref, bf_ref, ab_ref, bb_ref, m_ref, o_ref):
    x = x_ref[...]
    for j, l in enumerate(_FRONT):
        x = af_ref[j][None, :] * x + bf_ref[j][None, :] * _swap(x, _STRIDES[l])
    chunks = []
    for c in range(_N // 128):
        xc = x[:, c * 128:(c + 1) * 128]
        mc = m_ref[c * 128:(c + 1) * 128, :]
        chunks.append(jax.lax.dot_general(
            xc, mc,
            dimension_numbers=(((1,), (1,)), ((), ())),
            preferred_element_type=jnp.float32))
    x = jnp.concatenate(chunks, axis=-1)
    for j, l in enumerate(_BACK):
        x = ab_ref[j][None, :] * x + bb_ref[j][None, :] * _swap(x, _STRIDES[l])
    o_ref[...] = x


def kernel(x, values, idx_in, idx_out):
    del idx_in, idx_out
    batch, n = x.shape
    assert n == _N
    a_mid, b_mid, a_big, b_big = _extract_ab(values)

    amt = a_mid.T
    bmt = b_mid.T
    M = pl.pallas_call(
        _prep_body,
        out_shape=jax.ShapeDtypeStruct((n, 128), jnp.float32),
    )(amt, bmt)

    nf = len(_FRONT)
    af, ab = a_big[:nf], a_big[nf:]
    bf, bb = b_big[:nf], b_big[nf:]

    bt = 256
    if batch % bt:
        bt = batch
    nb = batch // bt
    return pl.pallas_call(
        _tile_body,
        grid=(nb,),
        in_specs=[
            pl.BlockSpec((bt, n), lambda i: (i, 0)),
            pl.BlockSpec((nf, n), lambda i: (0, 0)),
            pl.BlockSpec((nf, n), lambda i: (0, 0)),
            pl.BlockSpec((nf, n), lambda i: (0, 0)),
            pl.BlockSpec((nf, n), lambda i: (0, 0)),
            pl.BlockSpec((n, 128), lambda i: (0, 0)),
        ],
        out_specs=pl.BlockSpec((bt, n), lambda i: (i, 0)),
        out_shape=jax.ShapeDtypeStruct((batch, n), jnp.float32),
    )(x, af, bf, ab, bb, M)


# speedup vs baseline: 1.0128x; 1.0128x over previous
import jax
import jax.numpy as jnp
import numpy as np
from jax import lax
from jax.experimental import pallas as pl
from jax.experimental.pallas import tpu as pltpu

_N = 4096


def _butterfly_positions(n):
    indices_in = [[0, 0, 1, 1]]
    indices_out = [[0, 1, 0, 1]]
    curr_n = 2
    while curr_n < n:
        for i in range(len(indices_in)):
            indices_in[i] = indices_in[i] + [p + curr_n for p in indices_in[i]]
            indices_out[i] = indices_out[i] + [p + curr_n for p in indices_out[i]]
        sublist_low = list(range(curr_n)) * 2
        sublist_high = list(range(curr_n, curr_n * 2)) * 2
        new_idx_in = sublist_low + sublist_high
        indices_in.append(new_idx_in)
        new_idx_out = list(range(curr_n * 2)) * 2
        indices_out.append(new_idx_out)
        indices_in.insert(0, list(new_idx_in))
        indices_out.insert(0, list(new_idx_out))
        curr_n *= 2
    ii = np.array(indices_in)
    io = np.array(indices_out)
    L = ii.shape[0]
    pos_a = np.zeros((L, n), dtype=np.int64)
    pos_b = np.zeros((L, n), dtype=np.int64)
    strides = []
    for l in range(L):
        diag = ii[l] == io[l]
        pos_a[l, io[l, diag]] = np.nonzero(diag)[0]
        pos_b[l, io[l, ~diag]] = np.nonzero(~diag)[0]
        strides.append(int(abs(ii[l, ~diag][0] - io[l, ~diag][0])))
    return strides, pos_a, pos_b


_STRIDES, _POS_A, _POS_B = _butterfly_positions(_N)
_L = len(_STRIDES)
_MID = [l for l in range(_L) if _STRIDES[l] < 128]
_BIG = ([l for l in range(_L) if _STRIDES[l] >= 128 and l < _L // 2]
        + [l for l in range(_L) if _STRIDES[l] >= 128 and l > _L // 2])
_L128F = _MID[0] - 1
_L128B = _MID[-1] + 1
_FRONT_E = [l for l in _BIG if l < _L128F]
_BACK_E = [l for l in _BIG if l > _L128B]


def _selection_constants():
    sa = np.zeros((len(_MID), 256, 128), dtype=np.float32)
    sb = np.zeros((len(_MID), 256, 128), dtype=np.float32)
    for t, l in enumerate(_MID):
        for s_mat, pos in ((sa, _POS_A), (sb, _POS_B)):
            p = pos[l].reshape(32, 128)
            off = p - 256 * np.arange(32)[:, None]
            assert (off == off[0]).all()
            s_mat[t, off[0], np.arange(128)] = 1.0
    pa = np.zeros((len(_BIG), 32, 64), dtype=np.float32)
    pb = np.zeros((len(_BIG), 32, 64), dtype=np.float32)
    for t, l in enumerate(_BIG):
        for p_mat, pos in ((pa, _POS_A), (pb, _POS_B)):
            p = pos[l].reshape(32, 128)
            assert (p % 128 == np.arange(128)[None, :]).all()
            p_mat[t, np.arange(32), p[:, 0] // 128] = 1.0
    return sa, sb, pa, pb


_SA, _SB, _PA, _PB = _selection_constants()


def _extract_ab(values):
    vmid = values[_MID[0]:_MID[-1] + 1].reshape(len(_MID), 32, 256)
    dn = (((2,), (1,)), ((0,), (0,)))
    hp = lax.Precision.HIGHEST
    a_mid = lax.dot_general(vmid, jnp.asarray(_SA), dn, precision=hp,
                            preferred_element_type=jnp.float32)
    b_mid = lax.dot_general(vmid, jnp.asarray(_SB), dn, precision=hp,
                            preferred_element_type=jnp.float32)
    vbig = jnp.concatenate(
        [values[:_MID[0]], values[_MID[-1] + 1:]], axis=0).reshape(
            len(_BIG), 64, 128)
    a_big = lax.dot_general(jnp.asarray(_PA), vbig, dn, precision=hp,
                            preferred_element_type=jnp.float32)
    b_big = lax.dot_general(jnp.asarray(_PB), vbig, dn, precision=hp,
                            preferred_element_type=jnp.float32)
    return (a_mid.reshape(len(_MID), _N), b_mid.reshape(len(_MID), _N),
            a_big.reshape(len(_BIG), _N), b_big.reshape(len(_BIG), _N))


def _compose_body(amt_ref, bmt_ref, m_ref):
    row = lax.broadcasted_iota(jnp.int32, (_N, 128), 0)
    col = lax.broadcasted_iota(jnp.int32, (_N, 128), 1)
    m = jnp.where((row % 128) == col, 1.0, 0.0).astype(jnp.float32)
    for t, l in enumerate(_MID):
        s = _STRIDES[l]
        a = amt_ref[:, t:t + 1]
        b = bmt_ref[:, t:t + 1]
        take_lo = (row & s) == 0
        swapped = jnp.where(take_lo, jnp.roll(m, -s, axis=0),
                            jnp.roll(m, s, axis=0))
        m = a * m + b * swapped
    m_ref[...] = m


def _absorb_body(m_ref, ab4_ref, ab18t_ref, m1_ref, m2_ref):
    m1p_parts, m2p_parts = [], []
    for c in range(_N // 128):
        blk = m_ref[c * 128:(c + 1) * 128, :]
        m1p_parts.append(blk * ab4_ref[0, c * 128:(c + 1) * 128][None, :])
        m2p_parts.append(blk * ab4_ref[1, c * 128:(c + 1) * 128][None, :])
    m1p = jnp.concatenate(m1p_parts, axis=0)
    m2p = jnp.concatenate(m2p_parts, axis=0)
    a18 = ab18t_ref[:, 0:1]
    b18 = ab18t_ref[:, 1:2]
    row = lax.broadcasted_iota(jnp.int32, (_N, 128), 0)
    take_lo = (row & 128) == 0
    sw_m1 = jnp.where(take_lo, jnp.roll(m1p, -128, axis=0),
                      jnp.roll(m1p, 128, axis=0))
    sw_m2 = jnp.where(take_lo, jnp.roll(m2p, -128, axis=0),
                      jnp.roll(m2p, 128, axis=0))
    m1_ref[...] = a18 * m1p + b18 * sw_m2
    m2_ref[...] = a18 * m2p + b18 * sw_m1


def _butterfly_big(x, a_row, b_row, s):
    n = x.shape[-1]
    parts = []
    for g in range(n // (2 * s)):
        lo = x[:, g * 2 * s:g * 2 * s + s]
        hi = x[:, g * 2 * s + s:(g + 1) * 2 * s]
        a_lo = a_row[g * 2 * s:g * 2 * s + s][None, :]
        a_hi = a_row[g * 2 * s + s:(g + 1) * 2 * s][None, :]
        b_lo = b_row[g * 2 * s:g * 2 * s + s][None, :]
        b_hi = b_row[g * 2 * s + s:(g + 1) * 2 * s][None, :]
        parts.append(a_lo * lo + b_lo * hi)
        parts.append(a_hi * hi + b_hi * lo)
    return jnp.concatenate(parts, axis=-1)


def _tile_body(x_ref, af_ref, bf_ref, ab_ref, bb_ref, m1_scr, m2_scr, o_ref):
    x = x_ref[...]
    for j, l in enumerate(_FRONT_E):
        x = _butterfly_big(x, af_ref[j], bf_ref[j], _STRIDES[l])
    chunks = []
    dn = (((1,), (1,)), ((), ()))
    for c in range(_N // 128):
        xc = x[:, c * 128:(c + 1) * 128]
        xp = x[:, (c ^ 1) * 128:((c ^ 1) + 1) * 128]
        m1c = m1_scr[c * 128:(c + 1) * 128, :]
        m2c = m2_scr[c * 128:(c + 1) * 128, :]
        chunks.append(
            jax.lax.dot_general(xc, m1c, dimension_numbers=dn,
                                preferred_element_type=jnp.float32)
            + jax.lax.dot_general(xp, m2c, dimension_numbers=dn,
                                  preferred_element_type=jnp.float32))
    x = jnp.concatenate(chunks, axis=-1)
    for j, l in enumerate(_BACK_E):
        x = _butterfly_big(x, ab_ref[j], bb_ref[j], _STRIDES[l])
    o_ref[...] = x


def kernel(x, values, idx_in, idx_out):
    del idx_in, idx_out
    batch, n = x.shape
    assert n == _N
    a_mid, b_mid, a_big, b_big = _extract_ab(values)

    amt = a_mid.T
    bmt = b_mid.T
    nf = len(_FRONT_E)
    i4 = _BIG.index(_L128F)
    i18 = _BIG.index(_L128B)
    ab4 = jnp.stack([a_big[i4], b_big[i4]])
    ab18t = jnp.stack([a_big[i18], b_big[i18]], 1)
    af = a_big[:nf]
    bf = b_big[:nf]
    ab = a_big[i18 + 1:]
    bb = b_big[i18 + 1:]

    m = pl.pallas_call(
        _compose_body,
        out_shape=jax.ShapeDtypeStruct((n, 128), jnp.float32),
    )(amt, bmt)
    m1, m2 = pl.pallas_call(
        _absorb_body,
        out_shape=[jax.ShapeDtypeStruct((n, 128), jnp.float32),
                   jax.ShapeDtypeStruct((n, 128), jnp.float32)],
    )(m, ab4, ab18t)

    bt = 256
    if batch % bt:
        bt = batch
    nb = batch // bt
    return pl.pallas_call(
        _tile_body,
        grid=(nb,),
        in_specs=[
            pl.BlockSpec((bt, n), lambda i: (i, 0)),
            pl.BlockSpec((nf, n), lambda i: (0, 0)),
            pl.BlockSpec((nf, n), lambda i: (0, 0)),
            pl.BlockSpec((nf, n), lambda i: (0, 0)),
            pl.BlockSpec((nf, n), lambda i: (0, 0)),
            pl.BlockSpec((n, 128), lambda i: (0, 0)),
            pl.BlockSpec((n, 128), lambda i: (0, 0)),
        ],
        out_specs=pl.BlockSpec((bt, n), lambda i: (i, 0)),
        out_shape=jax.ShapeDtypeStruct((batch, n), jnp.float32),
    )(x, af, bf, ab, bb, m1, m2)
